# baseline (speedup 1.0000x reference)
import jax
import jax.numpy as jnp
from jax import lax
from jax.experimental import pallas as pl
from jax.experimental.pallas import tpu as pltpu, tpu_sc as plsc

B = 16384
D = 64
NNEG = 20
NC = 2
NS = 16
NW = NC * NS
BPW = B // NW
CH = 32
NCHUNK = BPW // CH
L = 16
GPC = CH // L


def _body(in_table, out_table, in_idx, ctx_idx, neg_idx, pos_out, neg_out,
          in_idx_v, ctx_idx_v, neg_raw_v, neg_idx_t, in_rows, pos_rows,
          neg_rows, pos_v, neg_v, sem):
    wid = lax.axis_index("s") * NC + lax.axis_index("c")

    pltpu.sync_copy(in_idx.at[pl.ds(wid * BPW, BPW)], in_idx_v)
    pltpu.sync_copy(ctx_idx.at[pl.ds(wid * BPW, BPW)], ctx_idx_v)
    pltpu.sync_copy(neg_idx.at[pl.ds(wid * BPW * NNEG, BPW * NNEG)],
                    neg_raw_v)

    iota = lax.iota(jnp.int32, L)

    def chunk_body(c, carry):
        base = c * (CH * NNEG)
        for j in range(NNEG):
            for g in range(GPC):
                fidx = base + (iota + g * L) * NNEG + j
                col = plsc.load_gather(neg_raw_v, [fidx])
                neg_idx_t[j, pl.ds(g * L, L)] = col

        cps = [
            pltpu.async_copy(in_table.at[in_idx_v.at[pl.ds(c * CH, CH)]],
                             in_rows, sem),
            pltpu.async_copy(out_table.at[ctx_idx_v.at[pl.ds(c * CH, CH)]],
                             pos_rows, sem),
        ]
        for j in range(NNEG):
            cps.append(pltpu.async_copy(out_table.at[neg_idx_t.at[j]],
                                        neg_rows.at[j], sem))
        for cp in cps:
            cp.wait()

        for g in range(GPC):
            rid = iota + (g * L)

            def d_body(d, accs):
                dvec = (iota + d) & (D - 1)
                inv = plsc.load_gather(in_rows, [rid, dvec])
                pv = plsc.load_gather(pos_rows, [rid, dvec])
                new = [accs[0] + inv * pv]
                for j in range(NNEG):
                    jvec = jnp.full((L,), j, jnp.int32)
                    nv = plsc.load_gather(neg_rows, [jvec, rid, dvec])
                    new.append(accs[j + 1] + inv * nv)
                return tuple(new)

            zeros = tuple(jnp.zeros((L,), jnp.float32)
                          for _ in range(NNEG + 1))
            accs = lax.fori_loop(0, D, d_body, zeros)

            off = c * CH + g * L
            pos_v[pl.ds(off, L)] = accs[0]
            widx = (iota + off) * NNEG
            for j in range(NNEG):
                plsc.store_scatter(neg_v, [widx + j], accs[j + 1])
        return carry

    lax.fori_loop(0, NCHUNK, chunk_body, 0)

    pltpu.sync_copy(pos_v, pos_out.at[pl.ds(wid * BPW, BPW)])
    pltpu.sync_copy(neg_v, neg_out.at[pl.ds(wid * BPW * NNEG, BPW * NNEG)])


@jax.jit
def _skipgram(in_table, out_table, in_idx, ctx_idx, neg_idx):
    mesh = plsc.VectorSubcoreMesh(core_axis_name="c", subcore_axis_name="s")
    f = pl.kernel(
        _body,
        out_type=[
            jax.ShapeDtypeStruct((B,), jnp.float32),
            jax.ShapeDtypeStruct((B * NNEG,), jnp.float32),
        ],
        mesh=mesh,
        scratch_types=[
            pltpu.VMEM((BPW,), jnp.int32),
            pltpu.VMEM((BPW,), jnp.int32),
            pltpu.VMEM((BPW * NNEG,), jnp.int32),
            pltpu.VMEM((NNEG, CH), jnp.int32),
            pltpu.VMEM((CH, D), jnp.float32),
            pltpu.VMEM((CH, D), jnp.float32),
            pltpu.VMEM((NNEG, CH, D), jnp.float32),
            pltpu.VMEM((BPW,), jnp.float32),
            pltpu.VMEM((BPW * NNEG,), jnp.float32),
            pltpu.SemaphoreType.DMA,
        ],
        compiler_params=pltpu.CompilerParams(use_tc_tiling_on_sc=False,
                                             needs_layout_passes=False),
    )
    return f(in_table, out_table, in_idx, ctx_idx, neg_idx)


def kernel(in_table, out_table, inputs, contexts, negatives):
    in_idx = inputs.reshape(B)
    ctx_idx = contexts.reshape(B)
    neg_idx = negatives.reshape(B * NNEG)
    pos, neg = _skipgram(in_table, out_table, in_idx, ctx_idx, neg_idx)
    return pos, neg.reshape(B, NNEG)
